# baseline (speedup 1.0000x reference)
import functools

import jax
import jax.numpy as jnp
from jax import lax
from jax.experimental import pallas as pl
from jax.experimental.pallas import tpu as pltpu
from jax.experimental.pallas import tpu_sc as plsc

B = 8
C = 3
H = 224
P = 4
HIDDEN = 256
D = 256
K = 1024
PD = C * P * P
HP = H // P
WP = 224 // P
N = B * HP * WP
COMMITMENT = 0.25
DATA_VAR = 1.0

TB_A = 256
NB_A = N // TB_A
TB_C = 512
NB_C = N // TB_C

SC_NW = 32
SC_ROWS_PER_W = N // SC_NW
SC_CHUNK = 112
SC_NCHUNK = SC_ROWS_PER_W // SC_CHUNK


def _patchify(x):
    b, c, h, w = x.shape
    x = x.reshape(b, c, h // P, P, w // P, P)
    return x.transpose(0, 2, 4, 1, 3, 5).reshape(b, h // P, w // P, c * P * P)


def _unpatchify(d):
    b, h, w, _ = d.shape
    return d.reshape(b, h, w, C, P, P).transpose(0, 3, 1, 4, 2, 5).reshape(b, C, h * P, w * P)


def _enc_vq_body(p_ref, we_ref, be_ref, wp_ref, bp_ref, emb_ref,
                 z_ref, idx_ref, counts_ref):
    i = pl.program_id(0)
    h = jnp.maximum(jnp.dot(p_ref[...], we_ref[...]) + be_ref[...], 0.0)
    z = jnp.dot(h, wp_ref[...]) + bp_ref[...]
    z_ref[...] = z
    emb = emb_ref[...]
    zsq = jnp.sum(z * z, axis=1, keepdims=True)
    esq = jnp.sum(emb * emb, axis=1)
    mm2 = lax.dot_general(2.0 * z, emb, (((1,), (1,)), ((), ())))
    dist = (zsq + esq) - mm2
    m = jnp.min(dist, axis=1, keepdims=True)
    ks = lax.broadcasted_iota(jnp.int32, dist.shape, 1)
    idx = jnp.min(jnp.where(dist == m, ks, K), axis=1).astype(jnp.int32)
    idx_ref[0, 0, :] = idx
    onehot = (idx[:, None] == lax.broadcasted_iota(jnp.int32, (TB_A, K), 1))
    part = jnp.sum(onehot.astype(jnp.float32), axis=0, keepdims=True)

    @pl.when(i == 0)
    def _():
        counts_ref[...] = jnp.zeros_like(counts_ref)

    counts_ref[...] += part


def _enc_vq(patches, W_enc, b_enc, W_pre, b_pre, emb):
    return pl.pallas_call(
        _enc_vq_body,
        grid=(NB_A,),
        in_specs=[
            pl.BlockSpec((TB_A, PD), lambda i: (i, 0)),
            pl.BlockSpec((PD, HIDDEN), lambda i: (0, 0)),
            pl.BlockSpec((1, HIDDEN), lambda i: (0, 0)),
            pl.BlockSpec((HIDDEN, D), lambda i: (0, 0)),
            pl.BlockSpec((1, D), lambda i: (0, 0)),
            pl.BlockSpec((K, D), lambda i: (0, 0)),
        ],
        out_specs=[
            pl.BlockSpec((TB_A, D), lambda i: (i, 0)),
            pl.BlockSpec((1, 1, TB_A), lambda i: (i, 0, 0)),
            pl.BlockSpec((1, K), lambda i: (0, 0)),
        ],
        out_shape=[
            jax.ShapeDtypeStruct((N, D), jnp.float32),
            jax.ShapeDtypeStruct((NB_A, 1, TB_A), jnp.int32),
            jax.ShapeDtypeStruct((1, K), jnp.float32),
        ],
    )(patches, W_enc, b_enc, W_pre, b_pre, emb)


def _sc_gather_body(emb_hbm, idx_hbm, out_hbm, idx_v, rows_v, sem):
    c = lax.axis_index("c")
    s = lax.axis_index("s")
    wid = s * 2 + c
    base = wid * SC_ROWS_PER_W

    def chunk(i, carry):
        off = base + i * SC_CHUNK
        pltpu.sync_copy(idx_hbm.at[pl.ds(off, SC_CHUNK)], idx_v)
        pltpu.async_copy(emb_hbm.at[idx_v], rows_v, sem).wait()
        pltpu.sync_copy(rows_v, out_hbm.at[pl.ds(off, SC_CHUNK)])
        return carry

    lax.fori_loop(0, SC_NCHUNK, chunk, 0)


@functools.cache
def _sc_gather_kernel():
    return pl.kernel(
        _sc_gather_body,
        out_type=jax.ShapeDtypeStruct((N, D), jnp.float32),
        mesh=plsc.VectorSubcoreMesh(core_axis_name="c", subcore_axis_name="s"),
        scratch_types=[
            pltpu.VMEM((SC_CHUNK,), jnp.int32),
            pltpu.VMEM((SC_CHUNK, D), jnp.float32),
            pltpu.SemaphoreType.DMA,
        ],
    )


def _sc_gather(emb, idx):
    return _sc_gather_kernel()(emb, idx)


def _dec_loss_body(p_ref, z_ref, q_ref, wd_ref, bd_ref, counts_ref,
                   qst_ref, dec_ref, loss_ref, perp_ref, rec_ref, vq_ref,
                   acc_ref):
    i = pl.program_id(0)
    z = z_ref[...]
    q = q_ref[...]
    qst = z + (q - z)
    qst_ref[...] = qst
    dec = jnp.dot(qst, wd_ref[...]) + bd_ref[...]
    dec_ref[...] = dec

    @pl.when(i == 0)
    def _():
        acc_ref[0] = 0.0
        acc_ref[1] = 0.0

    acc_ref[0] += jnp.sum((q - z) ** 2)
    acc_ref[1] += jnp.sum((dec - p_ref[...]) ** 2)

    @pl.when(i == NB_C - 1)
    def _():
        latent = acc_ref[0] / (N * D)
        vq = latent + COMMITMENT * latent
        rec = acc_ref[1] / (N * PD) / DATA_VAR
        p = counts_ref[...] / N
        ent = jnp.sum(p * jnp.log(p + 1e-10))
        perp_ref[...] = jnp.exp(-ent).reshape(1, 1)
        vq_ref[...] = vq.reshape(1, 1)
        rec_ref[...] = rec.reshape(1, 1)
        loss_ref[...] = (rec + vq).reshape(1, 1)


def _dec_loss(patches, z, q, W_dec, b_dec, counts):
    return pl.pallas_call(
        _dec_loss_body,
        grid=(NB_C,),
        in_specs=[
            pl.BlockSpec((TB_C, PD), lambda i: (i, 0)),
            pl.BlockSpec((TB_C, D), lambda i: (i, 0)),
            pl.BlockSpec((TB_C, D), lambda i: (i, 0)),
            pl.BlockSpec((D, PD), lambda i: (0, 0)),
            pl.BlockSpec((1, PD), lambda i: (0, 0)),
            pl.BlockSpec((1, K), lambda i: (0, 0)),
        ],
        out_specs=[
            pl.BlockSpec((TB_C, D), lambda i: (i, 0)),
            pl.BlockSpec((TB_C, PD), lambda i: (i, 0)),
            pl.BlockSpec((1, 1), lambda i: (0, 0)),
            pl.BlockSpec((1, 1), lambda i: (0, 0)),
            pl.BlockSpec((1, 1), lambda i: (0, 0)),
            pl.BlockSpec((1, 1), lambda i: (0, 0)),
        ],
        out_shape=[
            jax.ShapeDtypeStruct((N, D), jnp.float32),
            jax.ShapeDtypeStruct((N, PD), jnp.float32),
            jax.ShapeDtypeStruct((1, 1), jnp.float32),
            jax.ShapeDtypeStruct((1, 1), jnp.float32),
            jax.ShapeDtypeStruct((1, 1), jnp.float32),
            jax.ShapeDtypeStruct((1, 1), jnp.float32),
        ],
        scratch_shapes=[pltpu.SMEM((2,), jnp.float32)],
    )(patches, z, q, W_dec, b_dec, counts)


def kernel(inputs, W_enc, b_enc, W_pre, b_pre, W_dec, b_dec, emb):
    patches = _patchify(inputs).reshape(N, PD)
    z, idx3, counts = _enc_vq(patches, W_enc, b_enc.reshape(1, HIDDEN),
                              W_pre, b_pre.reshape(1, D), emb)
    idx = idx3.reshape(N)
    q = _sc_gather(emb, idx)
    qst, dec, loss, perp, rec, vq = _dec_loss(
        patches, z, q, W_dec, b_dec.reshape(1, PD), counts)
    x_rec = _unpatchify(dec.reshape(B, HP, WP, PD))
    return (loss.reshape(()), x_rec, qst.reshape(B, HP, WP, D),
            perp.reshape(()), rec.reshape(()), vq.reshape(()))


# speedup vs baseline: 1.0471x; 1.0471x over previous
import functools

import jax
import jax.numpy as jnp
from jax import lax
from jax.experimental import pallas as pl
from jax.experimental.pallas import tpu as pltpu
from jax.experimental.pallas import tpu_sc as plsc

B = 8
C = 3
H = 224
P = 4
HIDDEN = 256
D = 256
K = 1024
PD = C * P * P
HP = H // P
WP = 224 // P
N = B * HP * WP
COMMITMENT = 0.25
DATA_VAR = 1.0

TB_A = 256
NB_A = N // TB_A
TB_C = 512
NB_C = N // TB_C

SC_NW = 32
SC_ROWS_PER_W = N // SC_NW
SC_CHUNK = 112
SC_NCHUNK = SC_ROWS_PER_W // SC_CHUNK


def _patchify(x):
    eye = jnp.eye(PD, dtype=x.dtype).reshape(PD, C, P, P)
    dn = lax.conv_dimension_numbers(x.shape, eye.shape, ("NCHW", "OIHW", "NHWC"))
    return lax.conv_general_dilated(x, eye, (P, P), "VALID", dimension_numbers=dn)


def _unpatchify(d):
    eye = jnp.eye(PD, dtype=d.dtype).reshape(C, P, P, PD).transpose(1, 2, 3, 0)
    eye = eye[::-1, ::-1]
    return lax.conv_transpose(d, eye, (P, P), "VALID",
                              dimension_numbers=("NHWC", "HWIO", "NCHW"))


def _enc_vq_body(p_ref, we_ref, be_ref, wp_ref, bp_ref, emb_ref,
                 z_ref, idx_ref, counts_ref):
    i = pl.program_id(0)
    h = jnp.maximum(jnp.dot(p_ref[...], we_ref[...]) + be_ref[...], 0.0)
    z = jnp.dot(h, wp_ref[...]) + bp_ref[...]
    z_ref[...] = z
    emb = emb_ref[...]
    zsq = jnp.sum(z * z, axis=1, keepdims=True)
    esq = jnp.sum(emb * emb, axis=1)
    mm2 = lax.dot_general(2.0 * z, emb, (((1,), (1,)), ((), ())))
    dist = (zsq + esq) - mm2
    m = jnp.min(dist, axis=1, keepdims=True)
    ks = lax.broadcasted_iota(jnp.int32, dist.shape, 1)
    idx = jnp.min(jnp.where(dist == m, ks, K), axis=1).astype(jnp.int32)
    idx_ref[0, 0, :] = idx
    onehot = (idx[:, None] == lax.broadcasted_iota(jnp.int32, (TB_A, K), 1))
    part = jnp.sum(onehot.astype(jnp.float32), axis=0, keepdims=True)

    @pl.when(i == 0)
    def _():
        counts_ref[...] = jnp.zeros_like(counts_ref)

    counts_ref[...] += part


def _enc_vq(patches, W_enc, b_enc, W_pre, b_pre, emb):
    return pl.pallas_call(
        _enc_vq_body,
        grid=(NB_A,),
        in_specs=[
            pl.BlockSpec((TB_A, PD), lambda i: (i, 0)),
            pl.BlockSpec((PD, HIDDEN), lambda i: (0, 0)),
            pl.BlockSpec((1, HIDDEN), lambda i: (0, 0)),
            pl.BlockSpec((HIDDEN, D), lambda i: (0, 0)),
            pl.BlockSpec((1, D), lambda i: (0, 0)),
            pl.BlockSpec((K, D), lambda i: (0, 0)),
        ],
        out_specs=[
            pl.BlockSpec((TB_A, D), lambda i: (i, 0)),
            pl.BlockSpec((1, 1, TB_A), lambda i: (i, 0, 0)),
            pl.BlockSpec((1, K), lambda i: (0, 0)),
        ],
        out_shape=[
            jax.ShapeDtypeStruct((N, D), jnp.float32),
            jax.ShapeDtypeStruct((NB_A, 1, TB_A), jnp.int32),
            jax.ShapeDtypeStruct((1, K), jnp.float32),
        ],
    )(patches, W_enc, b_enc, W_pre, b_pre, emb)


def _sc_gather_body(emb_hbm, idx_hbm, out_hbm, idx_v, rows_v, sem):
    c = lax.axis_index("c")
    s = lax.axis_index("s")
    wid = s * 2 + c
    base = wid * SC_ROWS_PER_W

    def chunk(i, carry):
        off = base + i * SC_CHUNK
        pltpu.sync_copy(idx_hbm.at[pl.ds(off, SC_CHUNK)], idx_v)
        pltpu.async_copy(emb_hbm.at[idx_v], rows_v, sem).wait()
        pltpu.sync_copy(rows_v, out_hbm.at[pl.ds(off, SC_CHUNK)])
        return carry

    lax.fori_loop(0, SC_NCHUNK, chunk, 0)


@functools.cache
def _sc_gather_kernel():
    return pl.kernel(
        _sc_gather_body,
        out_type=jax.ShapeDtypeStruct((N, D), jnp.float32),
        mesh=plsc.VectorSubcoreMesh(core_axis_name="c", subcore_axis_name="s"),
        scratch_types=[
            pltpu.VMEM((SC_CHUNK,), jnp.int32),
            pltpu.VMEM((SC_CHUNK, D), jnp.float32),
            pltpu.SemaphoreType.DMA,
        ],
    )


def _sc_gather(emb, idx):
    return _sc_gather_kernel()(emb, idx)


def _dec_loss_body(p_ref, z_ref, q_ref, wd_ref, bd_ref, counts_ref,
                   qst_ref, dec_ref, loss_ref, perp_ref, rec_ref, vq_ref,
                   acc_ref):
    i = pl.program_id(0)
    z = z_ref[...]
    q = q_ref[...]
    qst = z + (q - z)
    qst_ref[...] = qst
    dec = jnp.dot(qst, wd_ref[...]) + bd_ref[...]
    dec_ref[...] = dec

    @pl.when(i == 0)
    def _():
        acc_ref[0] = 0.0
        acc_ref[1] = 0.0

    acc_ref[0] += jnp.sum((q - z) ** 2)
    acc_ref[1] += jnp.sum((dec - p_ref[...]) ** 2)

    @pl.when(i == NB_C - 1)
    def _():
        latent = acc_ref[0] / (N * D)
        vq = latent + COMMITMENT * latent
        rec = acc_ref[1] / (N * PD) / DATA_VAR
        p = counts_ref[...] / N
        ent = jnp.sum(p * jnp.log(p + 1e-10))
        perp_ref[...] = jnp.exp(-ent).reshape(1, 1)
        vq_ref[...] = vq.reshape(1, 1)
        rec_ref[...] = rec.reshape(1, 1)
        loss_ref[...] = (rec + vq).reshape(1, 1)


def _dec_loss(patches, z, q, W_dec, b_dec, counts):
    return pl.pallas_call(
        _dec_loss_body,
        grid=(NB_C,),
        in_specs=[
            pl.BlockSpec((TB_C, PD), lambda i: (i, 0)),
            pl.BlockSpec((TB_C, D), lambda i: (i, 0)),
            pl.BlockSpec((TB_C, D), lambda i: (i, 0)),
            pl.BlockSpec((D, PD), lambda i: (0, 0)),
            pl.BlockSpec((1, PD), lambda i: (0, 0)),
            pl.BlockSpec((1, K), lambda i: (0, 0)),
        ],
        out_specs=[
            pl.BlockSpec((TB_C, D), lambda i: (i, 0)),
            pl.BlockSpec((TB_C, PD), lambda i: (i, 0)),
            pl.BlockSpec((1, 1), lambda i: (0, 0)),
            pl.BlockSpec((1, 1), lambda i: (0, 0)),
            pl.BlockSpec((1, 1), lambda i: (0, 0)),
            pl.BlockSpec((1, 1), lambda i: (0, 0)),
        ],
        out_shape=[
            jax.ShapeDtypeStruct((N, D), jnp.float32),
            jax.ShapeDtypeStruct((N, PD), jnp.float32),
            jax.ShapeDtypeStruct((1, 1), jnp.float32),
            jax.ShapeDtypeStruct((1, 1), jnp.float32),
            jax.ShapeDtypeStruct((1, 1), jnp.float32),
            jax.ShapeDtypeStruct((1, 1), jnp.float32),
        ],
        scratch_shapes=[pltpu.SMEM((2,), jnp.float32)],
    )(patches, z, q, W_dec, b_dec, counts)


def kernel(inputs, W_enc, b_enc, W_pre, b_pre, W_dec, b_dec, emb):
    patches = _patchify(inputs).reshape(N, PD)
    z, idx3, counts = _enc_vq(patches, W_enc, b_enc.reshape(1, HIDDEN),
                              W_pre, b_pre.reshape(1, D), emb)
    idx = idx3.reshape(N)
    q = _sc_gather(emb, idx)
    qst, dec, loss, perp, rec, vq = _dec_loss(
        patches, z, q, W_dec, b_dec.reshape(1, PD), counts)
    x_rec = _unpatchify(dec.reshape(B, HP, WP, PD))
    return (loss.reshape(()), x_rec, qst.reshape(B, HP, WP, D),
            perp.reshape(()), rec.reshape(()), vq.reshape(()))
